# baseline (speedup 1.0000x reference)
import jax
import jax.numpy as jnp
from jax import lax
from jax.experimental import pallas as pl
from jax.experimental.pallas import tpu as pltpu

_BLK = 2048
_NEG = float("-inf")


def _dot(x, w):
    return lax.dot_general(x, w, (((1,), (0,)), ((), ())),
                           preferred_element_type=jnp.float32)


def _dot0(x, y):
    return lax.dot_general(x, y, (((0,), (0,)), ((), ())),
                           preferred_element_type=jnp.float32)


def _gelu(x):
    return 0.5 * x * (1.0 + lax.erf(x * 0.7071067811865476))


def _to_col(r):
    i0 = lax.broadcasted_iota(jnp.int32, (64, 64), 0)
    i1 = lax.broadcasted_iota(jnp.int32, (64, 64), 1)
    eyef = (i0 == i1).astype(jnp.float32)
    return jnp.sum(eyef * r, axis=1, keepdims=True)


def _fused(p4_ref, feat_ref, bid_ref,
           lng_ref, lnb_ref,
           spW1_ref, spb1_ref, spW2_ref, spb2_ref, spW3_ref, spb3_ref,
           fw1_ref, fmb1_ref, fmW2_ref, fmb2_ref, fmW3_ref, fmb3_ref,
           amW1_ref, amb1_ref, amW2_ref, amb2_ref,
           amW3_ref, amb3_ref, amW4t_ref,
           tok_ref, cen_ref, msk_ref,
           m_scr, l_scr, acc_scr, accp_scr):
    i = pl.program_id(0)
    nb = pl.num_programs(0)

    @pl.when(i == 0)
    def _init():
        m_scr[...] = jnp.full(m_scr.shape, _NEG, jnp.float32)
        l_scr[...] = jnp.zeros(l_scr.shape, jnp.float32)
        acc_scr[...] = jnp.zeros(acc_scr.shape, jnp.float32)
        accp_scr[...] = jnp.zeros(accp_scr.shape, jnp.float32)

    f = feat_ref[...]
    p = p4_ref[...]
    bid = bid_ref[...]

    z = _dot(f, fw1_ref[...])
    h = _gelu(z[:, 0:256] + fmb1_ref[...])
    g2 = _gelu(_dot(h, fmW2_ref[...]) + fmb2_ref[...])

    mu = jnp.mean(p, axis=1, keepdims=True)
    var = jnp.mean((p - mu) ** 2, axis=1, keepdims=True)
    sp0 = (p - mu) * lax.rsqrt(var + 1e-5) * lng_ref[...] + lnb_ref[...]
    s = _gelu(_dot(sp0, spW1_ref[...]) + spb1_ref[...])
    s = _gelu(_dot(s, spW2_ref[...]) + spb2_ref[...])
    s = _dot(s, spW3_ref[...]) + spb3_ref[...]

    a = _gelu(z[:, 256:512] + _dot(s, amW1_ref[256:, :]) + amb1_ref[...])
    a = _gelu(_dot(a, amW2_ref[...]) + amb2_ref[...])
    a = _gelu(_dot(a, amW3_ref[...]) + amb3_ref[...])
    attn_t = _dot(a, amW4t_ref[...])

    batch_of_col = lax.broadcasted_iota(jnp.int32, attn_t.shape, 1) // 8
    onehot = bid == batch_of_col

    masked = jnp.where(onehot, attn_t, _NEG)
    m_blk = jnp.max(masked, axis=0, keepdims=True)
    m_old = m_scr[...]
    m_new = jnp.maximum(m_old, m_blk)
    m_safe = jnp.where(m_new == _NEG, 0.0, m_new)
    scale = jnp.where(m_old == _NEG, 0.0, jnp.exp(m_old - m_safe))
    G = jnp.where(onehot, jnp.exp(attn_t - m_safe), 0.0)

    l_scr[...] = l_scr[...] * scale + jnp.sum(G, axis=0, keepdims=True)
    scale_col = _to_col(scale)
    acc_scr[...] = acc_scr[...] * scale_col + _dot0(G, g2)
    accp_scr[...] = accp_scr[...] * scale_col + _dot0(G, p)
    m_scr[...] = m_new

    @pl.when(i == nb - 1)
    def _fin():
        l = l_scr[...]
        l_col = _to_col(l)
        denom = l_col * (1.0 + 1e-6)
        num = _dot(acc_scr[...], fmW3_ref[...]) + l_col * fmb3_ref[...]
        tok_ref[...] = jnp.where(l_col > 0, num / denom, 0.0)
        cen_ref[...] = jnp.where(l_col > 0, accp_scr[...] / denom, 0.0)
        msk_ref[...] = l > 0.0


def kernel(coords, features, batch_ids, times, ln_g, ln_b,
           sp_W1, sp_b1, sp_W2, sp_b2, sp_W3, sp_b3,
           fm_W1, fm_b1, fm_W2, fm_b2, fm_W3, fm_b3,
           am_W1, am_b1, am_W2, am_b2, am_W3, am_b3, am_W4, am_b4):
    N, FD = features.shape
    S = am_W4.shape[-1]
    TD = fm_W3.shape[-1]
    HD = fm_W2.shape[-1]
    B = 8
    nblk = N // _BLK

    points4 = jnp.concatenate([coords[:, :3], times[:, :1]], axis=-1)
    bid2 = batch_ids.astype(jnp.int32).reshape(N, 1)
    row = lambda v: v.reshape(1, -1)
    amW4t = jnp.tile(am_W4, (1, B))

    fw1 = jnp.concatenate([fm_W1, am_W1[:FD]], axis=1)
    weights = (row(ln_g), row(ln_b),
               sp_W1, row(sp_b1), sp_W2, row(sp_b2), sp_W3, row(sp_b3),
               fw1, row(fm_b1), fm_W2, row(fm_b2), fm_W3, row(fm_b3),
               am_W1, row(am_b1), am_W2, row(am_b2),
               am_W3, row(am_b3), amW4t)

    in_specs = [
        pl.BlockSpec((_BLK, 4), lambda i: (i, 0)),
        pl.BlockSpec((_BLK, FD), lambda i: (i, 0)),
        pl.BlockSpec((_BLK, 1), lambda i: (i, 0)),
    ] + [pl.BlockSpec(w.shape, lambda i: (0, 0)) for w in weights]
    out_specs = [
        pl.BlockSpec((B * S, TD), lambda i: (0, 0)),
        pl.BlockSpec((B * S, 4), lambda i: (0, 0)),
        pl.BlockSpec((1, B * S), lambda i: (0, 0)),
    ]

    tok, cen, msk = pl.pallas_call(
        _fused,
        grid=(nblk,),
        in_specs=in_specs,
        out_specs=out_specs,
        out_shape=[
            jax.ShapeDtypeStruct((B * S, TD), jnp.float32),
            jax.ShapeDtypeStruct((B * S, 4), jnp.float32),
            jax.ShapeDtypeStruct((1, B * S), jnp.bool_),
        ],
        scratch_shapes=[
            pltpu.VMEM((1, B * S), jnp.float32),
            pltpu.VMEM((1, B * S), jnp.float32),
            pltpu.VMEM((B * S, HD), jnp.float32),
            pltpu.VMEM((B * S, 4), jnp.float32),
        ],
        compiler_params=pltpu.CompilerParams(
            allow_input_fusion=[True] * (3 + len(weights))),
    )(points4, features, bid2, *weights)

    return tok.reshape(B, S, TD), cen.reshape(B, S, 4), msk.reshape(B, S)


# speedup vs baseline: 1.0654x; 1.0654x over previous
import jax
import jax.numpy as jnp
from jax import lax
from jax.experimental import pallas as pl
from jax.experimental.pallas import tpu as pltpu

_BLK = 4096
_NEG = float("-inf")


def _dot(x, w):
    return lax.dot_general(x, w, (((1,), (0,)), ((), ())),
                           preferred_element_type=jnp.float32)


def _dot0(x, y):
    return lax.dot_general(x, y, (((0,), (0,)), ((), ())),
                           preferred_element_type=jnp.float32)


def _gelu(x):
    return 0.5 * x * (1.0 + lax.erf(x * 0.7071067811865476))


def _to_col(r):
    i0 = lax.broadcasted_iota(jnp.int32, (64, 64), 0)
    i1 = lax.broadcasted_iota(jnp.int32, (64, 64), 1)
    eyef = (i0 == i1).astype(jnp.float32)
    return jnp.sum(eyef * r, axis=1, keepdims=True)


def _fused(p4_ref, feat_ref, bid_ref,
           lng_ref, lnb_ref,
           spW1_ref, spb1_ref, spW2_ref, spb2_ref, spW3_ref, spb3_ref,
           fmW1_ref, fmb1_ref, fmW2_ref, fmb2_ref, fmW3_ref, fmb3_ref,
           amW1_ref, amb1_ref, amW2_ref, amb2_ref,
           amW3_ref, amb3_ref, amW4t_ref,
           tok_ref, cen_ref, msk_ref,
           m_scr, l_scr, acc_scr, accp_scr):
    i = pl.program_id(0)
    nb = pl.num_programs(0)

    @pl.when(i == 0)
    def _init():
        m_scr[...] = jnp.full(m_scr.shape, _NEG, jnp.float32)
        l_scr[...] = jnp.zeros(l_scr.shape, jnp.float32)
        acc_scr[...] = jnp.zeros(acc_scr.shape, jnp.float32)
        accp_scr[...] = jnp.zeros(accp_scr.shape, jnp.float32)

    f = feat_ref[...]
    p = p4_ref[...]
    bid = bid_ref[...]

    h = _gelu(_dot(f, fmW1_ref[...]) + fmb1_ref[...])
    g2 = _gelu(_dot(h, fmW2_ref[...]) + fmb2_ref[...])

    mu = jnp.mean(p, axis=1, keepdims=True)
    var = jnp.mean((p - mu) ** 2, axis=1, keepdims=True)
    sp0 = (p - mu) * lax.rsqrt(var + 1e-5) * lng_ref[...] + lnb_ref[...]
    s = _gelu(_dot(sp0, spW1_ref[...]) + spb1_ref[...])
    s = _gelu(_dot(s, spW2_ref[...]) + spb2_ref[...])
    s = _dot(s, spW3_ref[...]) + spb3_ref[...]

    fd = f.shape[1]
    a = _gelu(_dot(f, amW1_ref[0:fd, :]) + _dot(s, amW1_ref[fd:, :])
              + amb1_ref[...])
    a = _gelu(_dot(a, amW2_ref[...]) + amb2_ref[...])
    a = _gelu(_dot(a, amW3_ref[...]) + amb3_ref[...])
    attn_t = _dot(a, amW4t_ref[...])

    batch_of_col = lax.broadcasted_iota(jnp.int32, attn_t.shape, 1) // 8
    onehot = bid == batch_of_col

    masked = jnp.where(onehot, attn_t, _NEG)
    m_blk = jnp.max(masked, axis=0, keepdims=True)
    m_old = m_scr[...]
    m_new = jnp.maximum(m_old, m_blk)
    m_safe = jnp.where(m_new == _NEG, 0.0, m_new)
    scale = jnp.where(m_old == _NEG, 0.0, jnp.exp(m_old - m_safe))
    G = jnp.where(onehot, jnp.exp(attn_t - m_safe), 0.0)

    l_scr[...] = l_scr[...] * scale + jnp.sum(G, axis=0, keepdims=True)
    scale_col = _to_col(scale)
    acc_scr[...] = acc_scr[...] * scale_col + _dot0(G, g2)
    accp_scr[...] = accp_scr[...] * scale_col + _dot0(G, p)
    m_scr[...] = m_new

    @pl.when(i == nb - 1)
    def _fin():
        l = l_scr[...]
        l_col = _to_col(l)
        denom = l_col * (1.0 + 1e-6)
        num = _dot(acc_scr[...], fmW3_ref[...]) + l_col * fmb3_ref[...]
        tok_ref[...] = jnp.where(l_col > 0, num / denom, 0.0)
        cen_ref[...] = jnp.where(l_col > 0, accp_scr[...] / denom, 0.0)
        msk_ref[...] = l > 0.0


def kernel(coords, features, batch_ids, times, ln_g, ln_b,
           sp_W1, sp_b1, sp_W2, sp_b2, sp_W3, sp_b3,
           fm_W1, fm_b1, fm_W2, fm_b2, fm_W3, fm_b3,
           am_W1, am_b1, am_W2, am_b2, am_W3, am_b3, am_W4, am_b4):
    N, FD = features.shape
    S = am_W4.shape[-1]
    TD = fm_W3.shape[-1]
    HD = fm_W2.shape[-1]
    B = 8
    nblk = N // _BLK

    points4 = jnp.concatenate([coords[:, :3], times[:, :1]], axis=-1)
    bid2 = batch_ids.astype(jnp.int32).reshape(N, 1)
    row = lambda v: v.reshape(1, -1)
    amW4t = jnp.tile(am_W4, (1, B))

    weights = (row(ln_g), row(ln_b),
               sp_W1, row(sp_b1), sp_W2, row(sp_b2), sp_W3, row(sp_b3),
               fm_W1, row(fm_b1), fm_W2, row(fm_b2), fm_W3, row(fm_b3),
               am_W1, row(am_b1), am_W2, row(am_b2),
               am_W3, row(am_b3), amW4t)

    in_specs = [
        pl.BlockSpec((_BLK, 4), lambda i: (i, 0)),
        pl.BlockSpec((_BLK, FD), lambda i: (i, 0)),
        pl.BlockSpec((_BLK, 1), lambda i: (i, 0)),
    ] + [pl.BlockSpec(w.shape, lambda i: (0, 0)) for w in weights]
    out_specs = [
        pl.BlockSpec((B * S, TD), lambda i: (0, 0)),
        pl.BlockSpec((B * S, 4), lambda i: (0, 0)),
        pl.BlockSpec((1, B * S), lambda i: (0, 0)),
    ]

    tok, cen, msk = pl.pallas_call(
        _fused,
        grid=(nblk,),
        in_specs=in_specs,
        out_specs=out_specs,
        out_shape=[
            jax.ShapeDtypeStruct((B * S, TD), jnp.float32),
            jax.ShapeDtypeStruct((B * S, 4), jnp.float32),
            jax.ShapeDtypeStruct((1, B * S), jnp.bool_),
        ],
        scratch_shapes=[
            pltpu.VMEM((1, B * S), jnp.float32),
            pltpu.VMEM((1, B * S), jnp.float32),
            pltpu.VMEM((B * S, HD), jnp.float32),
            pltpu.VMEM((B * S, 4), jnp.float32),
        ],
        compiler_params=pltpu.CompilerParams(
            allow_input_fusion=[True] * (3 + len(weights))),
    )(points4, features, bid2, *weights)

    return tok.reshape(B, S, TD), cen.reshape(B, S, 4), msk.reshape(B, S)
